# baseline (speedup 1.0000x reference)
import functools

import jax
import jax.numpy as jnp
from jax import lax
from jax.experimental import pallas as pl
from jax.experimental.pallas import tpu as pltpu
from jax.experimental.pallas import tpu_sc as plsc

B = 4096
L = 200
D = 32
NW = 32
CHUNK = 1024
ITEMS = B * L // CHUNK
PER_W = ITEMS // NW
IDX_ROWS = CHUNK // 128

_mesh = plsc.VectorSubcoreMesh(core_axis_name="c", subcore_axis_name="s")


@functools.partial(
    pl.kernel,
    out_type=jax.ShapeDtypeStruct((B * L, D), jnp.float32),
    mesh=_mesh,
    scratch_types=[
        pltpu.VMEM((IDX_ROWS, 128), jnp.int32),
        pltpu.VMEM((CHUNK, D), jnp.float32),
        pltpu.VMEM((L, D), jnp.float32),
        pltpu.SemaphoreType.DMA,
    ],
    compiler_params=pltpu.CompilerParams(use_tc_tiling_on_sc=False),
)
def _emb_lookup(x_hbm, table_hbm, pos_hbm, out_hbm, idx_v, rows_v, pos_v, gsem):
    wid = lax.axis_index("s") * 2 + lax.axis_index("c")
    pltpu.sync_copy(pos_hbm, pos_v)

    def item_body(j, carry):
        m = wid * PER_W + j
        pltpu.sync_copy(x_hbm.at[pl.ds(m * IDX_ROWS, IDX_ROWS)], idx_v)
        descs = [
            pltpu.async_copy(table_hbm.at[idx_v.at[k]],
                             rows_v.at[pl.ds(k * 128, 128)], gsem)
            for k in range(IDX_ROWS)
        ]
        for d in descs:
            d.wait()

        l0 = lax.rem(m * CHUNK, L)

        def row_body(r, l):
            rows_v[r, pl.ds(0, 16)] = (
                rows_v[r, pl.ds(0, 16)] + pos_v[l, pl.ds(0, 16)])
            rows_v[r, pl.ds(16, 16)] = (
                rows_v[r, pl.ds(16, 16)] + pos_v[l, pl.ds(16, 16)])
            return lax.select(l == L - 1, 0, l + 1)

        lax.fori_loop(0, CHUNK, row_body, l0)

        pltpu.sync_copy(rows_v, out_hbm.at[pl.ds(m * CHUNK, CHUNK)])
        return carry

    lax.fori_loop(0, PER_W, item_body, 0)


def kernel(x, embedding_table, possitional_emb):
    xi = x.astype(jnp.int32).reshape(B * L // 128, 128)
    out = _emb_lookup(xi, embedding_table, possitional_emb)
    return out.reshape(B, L, D)
